# Initial kernel scaffold; baseline (speedup 1.0000x reference)
import jax
import jax.numpy as jnp
from jax.experimental import pallas as pl


def kernel(x, params):
    raise NotImplementedError("write your pallas kernel here")


# speedup vs baseline: 4.9428x; 4.9428x over previous
import functools

import jax
import jax.numpy as jnp
from jax import lax
from jax.experimental import pallas as pl
from jax.experimental.pallas import tpu as pltpu
from jax.experimental.pallas import tpu_sc as plsc

K = 20
N = 2048
B = 8
D_MODEL = 64; D_INNER = 128; D_STATE = 64; NHEADS = 2; HEADDIM = 64; D_CONV = 4
CONV_DIM = D_INNER + 2 * D_STATE

TILE = 256
NT = N // TILE


def _knn_body(xt_ref, xx_ref, xxr_ref, idx_ref):
    it = pl.program_id(1)
    Bfull = xt_ref[0]
    A = xt_ref[0, pl.ds(it * TILE, TILE), :]
    rown = xxr_ref[0]
    D = lax.dot_general(A.astype(jnp.bfloat16), Bfull.astype(jnp.bfloat16),
                        (((1,), (1,)), ((), ())),
                        preferred_element_type=jnp.float32)
    pd = (2.0 * D - rown) - xx_ref[0]
    s = lax.bitcast_convert_type(pd, jnp.int32)
    key = jnp.where(s < 0, s ^ jnp.int32(0x7FFFFFFF), s)
    col = lax.broadcasted_iota(jnp.int32, (TILE, N), 1)
    cols = []
    for _ in range(K):
        m = jnp.max(key, axis=1, keepdims=True)
        sel = jnp.where(key == m, col, jnp.int32(N))
        j = jnp.min(sel, axis=1, keepdims=True)
        cols.append(j)
        key = jnp.where(col == j, jnp.int32(-2147483648), key)
    idx_ref[0] = jnp.concatenate(cols, axis=1)


def _knn_call(xt, xx):
    Dp = xt.shape[-1]
    return pl.pallas_call(
        _knn_body,
        grid=(B, NT),
        in_specs=[
            pl.BlockSpec((1, N, Dp), lambda b, t: (b, 0, 0)),
            pl.BlockSpec((1, 1, N), lambda b, t: (b, 0, 0)),
            pl.BlockSpec((1, TILE, 1), lambda b, t: (b, t, 0)),
        ],
        out_specs=pl.BlockSpec((1, TILE, K), lambda b, t: (b, t, 0)),
        out_shape=jax.ShapeDtypeStruct((B, N, K), jnp.int32),
        compiler_params=pltpu.CompilerParams(
            dimension_semantics=("arbitrary", "arbitrary")),
    )(xt, xx, jnp.swapaxes(xx, 1, 2))


def _make_gather(Dp):
    NW = 32
    ROWS_W = B * N * K // NW
    CH = 128
    NCH = ROWS_W // CH
    mesh = plsc.VectorSubcoreMesh(core_axis_name="c", subcore_axis_name="s")

    @functools.partial(
        pl.kernel, mesh=mesh,
        out_type=jax.ShapeDtypeStruct((B * N * K, Dp), jnp.float32),
        scratch_types=[
            pltpu.VMEM((ROWS_W,), jnp.int32),
            pltpu.VMEM((CH, Dp), jnp.float32),
            pltpu.VMEM((CH, Dp), jnp.float32),
            pltpu.SemaphoreType.DMA,
            pltpu.SemaphoreType.DMA,
        ],
    )
    def gather_kernel(xt_hbm, idx_hbm, out_hbm, idx_v, buf0, buf1, sem0, sem1):
        wid = lax.axis_index("s") * 2 + lax.axis_index("c")
        b = wid // 4
        row0 = wid * ROWS_W
        pltpu.sync_copy(idx_hbm.at[pl.ds(row0, ROWS_W)], idx_v)
        table = xt_hbm.at[b]
        bufs = (buf0, buf1)
        sems = (sem0, sem1)
        handles = [None, None]
        handles[0] = pltpu.async_copy(
            table.at[idx_v.at[pl.ds(0, CH)]], buf0, sem0)
        for c in range(NCH):
            if c + 1 < NCH:
                handles[(c + 1) % 2] = pltpu.async_copy(
                    table.at[idx_v.at[pl.ds((c + 1) * CH, CH)]],
                    bufs[(c + 1) % 2], sems[(c + 1) % 2])
            handles[c % 2].wait()
            pltpu.sync_copy(bufs[c % 2], out_hbm.at[pl.ds(row0 + c * CH, CH)])

    return gather_kernel


def _conv_body(feat_ref, xt_ref, w_ref, m_ref, s_ref):
    Dp = xt_ref.shape[-1]
    F = feat_ref[...].reshape(TILE, K, Dp)
    xr = xt_ref[0]
    xr3 = xr[:, None, :]
    fm = F - xr3
    xrb = jnp.broadcast_to(xr3, (TILE, K, Dp))
    edge = jnp.concatenate([fm, xrb], axis=2)
    e2 = edge.reshape(TILE * K, 2 * Dp).astype(jnp.bfloat16)
    y = lax.dot_general(w_ref[...].astype(jnp.bfloat16), e2,
                        (((1,), (1,)), ((), ())),
                        preferred_element_type=jnp.float32)
    y3 = y.reshape(64, TILE, K)
    m_ref[0] = jnp.max(y3, axis=2)
    s1 = jnp.sum(y, axis=1, keepdims=True)
    s2 = jnp.sum(y * y, axis=1, keepdims=True)
    s_ref[0, 0] = jnp.concatenate([s1, s2], axis=1)


def _conv_call(feat, xt, w):
    Dp = xt.shape[-1]
    return pl.pallas_call(
        _conv_body,
        grid=(B, NT),
        in_specs=[
            pl.BlockSpec((TILE * K, Dp), lambda b, t: (b * NT + t, 0)),
            pl.BlockSpec((1, TILE, Dp), lambda b, t: (b, t, 0)),
            pl.BlockSpec((64, 2 * Dp), lambda b, t: (0, 0)),
        ],
        out_specs=[
            pl.BlockSpec((1, 64, TILE), lambda b, t: (b, 0, t)),
            pl.BlockSpec((1, 1, 64, 2), lambda b, t: (b, t, 0, 0)),
        ],
        out_shape=[
            jax.ShapeDtypeStruct((B, 64, N), jnp.float32),
            jax.ShapeDtypeStruct((B, NT, 64, 2), jnp.float32),
        ],
        compiler_params=pltpu.CompilerParams(
            dimension_semantics=("arbitrary", "arbitrary")),
    )(feat, xt, w)


def _edge_block(xt_pad, xx, p, i, d, gather_fn):
    Dp = xt_pad.shape[-1]
    W = p['W%d' % i]
    w = jnp.zeros((64, 2 * Dp), jnp.float32)
    w = w.at[:, :d].set(W[:, :d]).at[:, Dp:Dp + d].set(W[:, d:])
    idx = _knn_call(xt_pad, xx)
    feat = gather_fn(xt_pad, idx.reshape(-1))
    M, stats = _conv_call(feat, xt_pad, w)
    NBK = B * N * K
    s = stats.sum(axis=(0, 1))
    mean = s[:, 0] / NBK
    var = s[:, 1] / NBK - mean * mean
    pre = (M - mean[None, :, None]) / jnp.sqrt(var + 1e-5)[None, :, None] \
        * p['g%d' % i][None, :, None] + p['b%d' % i][None, :, None]
    pi = jnp.where(pre >= 0, pre, 0.2 * pre)
    return pi, pi.max(axis=2), pi.mean(axis=2)


def _leaky(v):
    return jnp.where(v >= 0, v, 0.2 * v)


def _bn_sm(y, gamma, beta, axes):
    m = y.mean(axis=axes, keepdims=True)
    v = y.var(axis=axes, keepdims=True)
    sh = [1] * y.ndim; sh[1] = -1
    return (y - m) / jnp.sqrt(v + 1e-5) * gamma.reshape(sh) + beta.reshape(sh)


def _mamba2(u, p, pref):
    bb, L, _ = u.shape
    zxbcdt = u @ p[pref + '_in_w'] + p[pref + '_in_b']
    z = zxbcdt[..., :D_INNER]
    xBC = zxbcdt[..., D_INNER:D_INNER + CONV_DIM]
    dt = zxbcdt[..., -NHEADS:]
    xp = jnp.pad(xBC, ((0, 0), (D_CONV - 1, 0), (0, 0)))
    stk = jnp.stack([xp[:, j:j + L, :] for j in range(D_CONV)], axis=0)
    xBC = jax.nn.silu(jnp.einsum('jblc,jc->blc', stk, p[pref + '_conv_w']) + p[pref + '_conv_b'])
    xh = xBC[..., :D_INNER].reshape(bb, L, NHEADS, HEADDIM)
    Bm = xBC[..., D_INNER:D_INNER + D_STATE]
    Cm = xBC[..., D_INNER + D_STATE:]
    dt = jax.nn.softplus(dt + p[pref + '_dt_bias'])
    A = -jnp.exp(p[pref + '_A_log'])
    dA = jnp.exp(dt * A)
    def step(h, inp):
        dA_t, dt_t, x_t, B_t, C_t = inp
        h = dA_t[:, :, None, None] * h + (dt_t[:, :, None] * x_t)[..., None] * B_t[:, None, None, :]
        y = jnp.einsum('bhps,bs->bhp', h, C_t)
        return h, y
    h0 = jnp.zeros((bb, NHEADS, HEADDIM, D_STATE))
    xs = (jnp.moveaxis(dA, 1, 0), jnp.moveaxis(dt, 1, 0), jnp.moveaxis(xh, 1, 0),
          jnp.moveaxis(Bm, 1, 0), jnp.moveaxis(Cm, 1, 0))
    _, ys = lax.scan(step, h0, xs)
    y = jnp.moveaxis(ys, 0, 1) + p[pref + '_D'][None, None, :, None] * xh
    y = y.reshape(bb, L, D_INNER)
    y = y * jax.nn.silu(z)
    y = y * lax.rsqrt(jnp.mean(y * y, axis=-1, keepdims=True) + 1e-5) * p[pref + '_norm_w']
    return y @ p[pref + '_out_w']


def _fuse(w, p, pref):
    y = _leaky(_bn_sm(jnp.einsum('oc,bcl->bol', p[pref + '_W1'], w), p[pref + '_g1'], p[pref + '_b1'], (0, 2)))
    y = _leaky(_bn_sm(jnp.einsum('oc,bcl->bol', p[pref + '_W2'], y), p[pref + '_g2'], p[pref + '_b2'], (0, 2)))
    return jnp.einsum('oc,bcl->bol', p[pref + '_W3'], y) + p[pref + '_b3'][None, :, None]


def _final_body(pt_ref, w5_ref, y_ref):
    P = pt_ref[0].astype(jnp.bfloat16)
    y_ref[0] = jnp.dot(w5_ref[...].astype(jnp.bfloat16), P,
                       preferred_element_type=jnp.float32)


def _final_conv_call(point, W5):
    return pl.pallas_call(
        _final_body,
        grid=(B, NT),
        in_specs=[
            pl.BlockSpec((1, 256, TILE), lambda b, t: (b, 0, t)),
            pl.BlockSpec((1024, 256), lambda b, t: (0, 0)),
        ],
        out_specs=pl.BlockSpec((1, 1024, TILE), lambda b, t: (b, 0, t)),
        out_shape=jax.ShapeDtypeStruct((B, 1024, N), jnp.float32),
        compiler_params=pltpu.CompilerParams(
            dimension_semantics=("arbitrary", "arbitrary")),
    )(point, W5)


def kernel(x, params):
    p = params
    gather128 = _make_gather(128)
    xt = jnp.swapaxes(x, 1, 2)
    xt16 = jnp.pad(xt, ((0, 0), (0, 0), (0, 125)))
    xx1 = jnp.sum(x * x, axis=1, keepdims=True)

    pi1, p1x, p1m = _edge_block(xt16, xx1, p, 1, 3, gather128)
    def cat(pi_cm):
        x_cm = jnp.concatenate([x, pi_cm], axis=1)
        xt_pad = jnp.concatenate([xt, jnp.swapaxes(pi_cm, 1, 2),
                                  jnp.zeros((B, N, 61), jnp.float32)], axis=-1)
        xx = jnp.sum(x_cm * x_cm, axis=1, keepdims=True)
        return xt_pad, xx
    a2 = cat(pi1)
    pi2, p2x, p2m = _edge_block(a2[0], a2[1], p, 2, 67, gather128)
    a3 = cat(pi2)
    pi3, p3x, p3m = _edge_block(a3[0], a3[1], p, 3, 67, gather128)
    a4 = cat(pi3)
    pi4, p4x, p4m = _edge_block(a4[0], a4[1], p, 4, 67, gather128)

    stack = jnp.stack([p1x, p2x, p3x, p4x], axis=1)
    w = _fuse(jnp.swapaxes(_mamba2(stack, p, 'm1'), 2, 1), p, 'f1')
    ws = [jnp.mean(w[:, 0, i], axis=0) for i in range(4)]
    stack2 = jnp.stack([p1m, p2m, p3m, p4m], axis=1)
    w2 = _fuse(jnp.swapaxes(_mamba2(stack2, p, 'm2'), 2, 1), p, 'f2')
    ws2 = [jnp.mean(w2[:, 0, i], axis=0) for i in range(4)]

    point = jnp.concatenate([
        (ws[0] + ws2[0]) * pi1,
        (ws[1] + ws2[1]) * pi2,
        (ws[2] + ws2[2]) * pi3,
        (ws[3] + ws2[3]) * pi4], axis=1)

    y5 = _final_conv_call(point, p['W5'])
    mean = y5.mean(axis=(0, 2), keepdims=True)
    var = y5.var(axis=(0, 2), keepdims=True)
    out = (y5 - mean) / jnp.sqrt(var + 1e-5) * p['g5'][None, :, None] + p['b5'][None, :, None]
    return _leaky(out)
